# baseline (speedup 1.0000x reference)
import jax
import jax.numpy as jnp
from jax import lax
from jax.experimental import pallas as pl
from jax.experimental.pallas import tpu as pltpu
from jax.experimental.pallas import tpu_sc as plsc

N = 10000
D = 128
E = 320000

NC = 2
NS = 16
NW = NC * NS

CHUNK = 128
NCHUNKS = 81
EPW = CHUNK * NCHUNKS
EP = EPW * NW

_LANE_DNUMS = lax.GatherDimensionNumbers(
    offset_dims=(), collapsed_slice_dims=(0,), start_index_map=(0,))


def _lane_broadcast(vec, j):
    idx = jnp.full((16, 1), j, dtype=jnp.int32)
    return lax.gather(vec, idx, _LANE_DNUMS, (1,),
                      mode=lax.GatherScatterMode.PROMISE_IN_BOUNDS)


def _agg_body(x_hbm, src_hbm, dst_hbm, w_hbm, z_hbm, out_hbm,
              sb0, sb1, sb2, wb0, wb1, wb2, db0, db1, db2,
              r0, r1, r2, acc_sh,
              gs0, gs1, gs2, ss0, ss1, ss2, es0, es1, es2, ds0, ds1, ds2):
    c = lax.axis_index("c")
    s = lax.axis_index("s")
    wid = s * NC + c
    SB = (sb0, sb1, sb2)
    WB = (wb0, wb1, wb2)
    DB = (db0, db1, db2)
    R = (r0, r1, r2)
    GS = (gs0, gs1, gs2)
    SS = (ss0, ss1, ss2)
    ES = (es0, es1, es2)
    DS = (ds0, ds1, ds2)

    def sw_start(k, b):
        pltpu.async_copy(src_hbm.at[wid, k], SB[b], ES[b])
        pltpu.async_copy(w_hbm.at[wid, k], WB[b], ES[b])

    def sw_wait(k, b):
        pltpu.make_async_copy(src_hbm.at[wid, k], SB[b], ES[b]).wait()
        pltpu.make_async_copy(w_hbm.at[wid, k], WB[b], ES[b]).wait()

    def d_start(k, b):
        pltpu.async_copy(dst_hbm.at[wid, k], DB[b], DS[b])

    def d_wait(k, b):
        pltpu.make_async_copy(dst_hbm.at[wid, k], DB[b], DS[b]).wait()

    def g_start(k, b):
        pltpu.async_copy(x_hbm.at[SB[b].at[0]], R[b], GS[b])

    def g_wait(k, b):
        pltpu.make_async_copy(x_hbm.at[SB[b].at[0]], R[b], GS[b]).wait()

    def s_start(k, b):
        pltpu.async_copy(R[b], acc_sh.at[DB[b].at[0]], SS[b], add=True)

    def s_wait(k, b):
        pltpu.make_async_copy(R[b], acc_sh.at[DB[b].at[0]], SS[b]).wait()

    def mul(k, b):
        def mul_group(g, c2):
            wv = WB[b][0, pl.ds(g * 16, 16)]
            for j in range(16):
                wb = _lane_broadcast(wv, j)
                e = g * 16 + j
                for d in range(8):
                    sl = pl.ds(d * 16, 16)
                    R[b][e, sl] = R[b][e, sl] * wb
            return c2
        lax.fori_loop(0, CHUNK // 16, mul_group, 0)

    @pl.when(s < 15)
    def _():
        pltpu.sync_copy(z_hbm.at[pl.ds(0, 624)],
                        acc_sh.at[pl.ds(s * 624, 624)])

    @pl.when(s == 15)
    def _():
        pltpu.sync_copy(z_hbm, acc_sh.at[pl.ds(15 * 624, 640)])

    plsc.subcore_barrier()

    sw_start(0, 0)
    sw_start(1, 1)
    sw_start(2, 2)
    d_start(0, 0)
    sw_wait(0, 0)
    g_start(0, 0)

    def outer(i, carry):
        k0 = i * 3
        for b in range(3):
            k = k0 + b
            bn = (b + 1) % 3
            g_wait(k, b)

            @pl.when(k >= 2)
            def _(k=k, bn=bn):
                s_wait(k - 2, bn)

            @pl.when(k < NCHUNKS - 1)
            def _(k=k, bn=bn):
                d_start(k + 1, bn)
                sw_wait(k + 1, bn)
                g_start(k + 1, bn)

            mul(k, b)

            @pl.when(k < NCHUNKS - 3)
            def _(k=k, b=b):
                sw_start(k + 3, b)

            d_wait(k, b)
            s_start(k, b)
        return carry

    lax.fori_loop(0, NCHUNKS // 3, outer, 0)
    s_wait(NCHUNKS - 2, (NCHUNKS - 2) % 3)
    s_wait(NCHUNKS - 1, (NCHUNKS - 1) % 3)

    plsc.subcore_barrier()

    @pl.when(s < 15)
    def _():
        pltpu.sync_copy(acc_sh.at[pl.ds(s * 624, 624)],
                        out_hbm.at[c, pl.ds(s * 624, 624)])

    @pl.when(s == 15)
    def _():
        pltpu.sync_copy(acc_sh.at[pl.ds(15 * 624, 640)],
                        out_hbm.at[c, pl.ds(15 * 624, 640)])


_agg_call = pl.kernel(
    _agg_body,
    out_type=jax.ShapeDtypeStruct((NC, N, D), jnp.float32),
    mesh=plsc.VectorSubcoreMesh(core_axis_name="c", subcore_axis_name="s"),
    scratch_types=(
        [pltpu.VMEM((1, CHUNK), jnp.int32) for _ in range(3)]
        + [pltpu.VMEM((1, CHUNK), jnp.float32) for _ in range(3)]
        + [pltpu.VMEM((1, CHUNK), jnp.int32) for _ in range(3)]
        + [pltpu.VMEM((CHUNK, D), jnp.float32) for _ in range(3)]
        + [pltpu.VMEM_SHARED((N, D), jnp.float32)]
        + [pltpu.SemaphoreType.DMA for _ in range(12)]
    ),
)


def _dense(p, W, b, relu):
    def body(p_ref, w_ref, b_ref, o_ref):
        acc = p_ref[0] + p_ref[1]
        r = jnp.dot(acc, w_ref[...], preferred_element_type=jnp.float32,
                    precision=lax.Precision.HIGHEST) + b_ref[...]
        o_ref[...] = jnp.maximum(r, 0.0) if relu else r

    R = 1000
    return pl.pallas_call(
        body,
        grid=(N // R,),
        in_specs=[
            pl.BlockSpec((2, R, D), lambda i: (0, i, 0)),
            pl.BlockSpec((D, D), lambda i: (0, 0)),
            pl.BlockSpec((1, D), lambda i: (0, 0)),
        ],
        out_specs=pl.BlockSpec((R, D), lambda i: (i, 0)),
        out_shape=jax.ShapeDtypeStruct((N, D), jnp.float32),
    )(p, W, b.reshape(1, D))


def kernel(x, edge_index, edge_weight, W1, b1, W2, b2):
    src = edge_index[0].astype(jnp.int32)
    dst = edge_index[1].astype(jnp.int32)
    w = edge_weight.astype(jnp.float32)
    pad = EP - E
    spread = (jnp.arange(pad, dtype=jnp.int32) * 37) % N
    src_p = jnp.concatenate([src, spread]).reshape(NW, NCHUNKS, 1, CHUNK)
    dst_p = jnp.concatenate([dst, spread]).reshape(NW, NCHUNKS, 1, CHUNK)
    w_p = jnp.pad(w, (0, pad)).reshape(NW, NCHUNKS, 1, CHUNK)
    zeros = jnp.zeros((640, D), jnp.float32)

    p1 = _agg_call(x, src_p, dst_p, w_p, zeros)
    h = _dense(p1, W1, b1, relu=True)
    p2 = _agg_call(h, src_p, dst_p, w_p, zeros)
    return _dense(p2, W2, b2, relu=False)


# speedup vs baseline: 1.0751x; 1.0751x over previous
import jax
import jax.numpy as jnp
from jax import lax
from jax.experimental import pallas as pl
from jax.experimental.pallas import tpu as pltpu
from jax.experimental.pallas import tpu_sc as plsc

N = 10000
D = 128
E = 320000

NC = 2
NS = 16
NW = NC * NS

CHUNK = 96
NCHUNKS = 108
EPW = CHUNK * NCHUNKS
EP = EPW * NW
NB = 4

_LANE_DNUMS = lax.GatherDimensionNumbers(
    offset_dims=(), collapsed_slice_dims=(0,), start_index_map=(0,))


def _lane_broadcast(vec, j):
    idx = jnp.full((16, 1), j, dtype=jnp.int32)
    return lax.gather(vec, idx, _LANE_DNUMS, (1,),
                      mode=lax.GatherScatterMode.PROMISE_IN_BOUNDS)


def _agg_body(x_hbm, src_hbm, dst_hbm, w_hbm, out_hbm,
              sb0, sb1, sb2, sb3, wb0, wb1, wb2, wb3,
              db0, db1, db2, db3, r0, r1, r2, r3, acc_sh,
              gs0, gs1, gs2, gs3, ss0, ss1, ss2, ss3,
              es0, es1, es2, es3, ds0, ds1, ds2, ds3):
    c = lax.axis_index("c")
    s = lax.axis_index("s")
    wid = s * NC + c
    SB = (sb0, sb1, sb2, sb3)
    WB = (wb0, wb1, wb2, wb3)
    DB = (db0, db1, db2, db3)
    R = (r0, r1, r2, r3)
    GS = (gs0, gs1, gs2, gs3)
    SS = (ss0, ss1, ss2, ss3)
    ES = (es0, es1, es2, es3)
    DS = (ds0, ds1, ds2, ds3)

    def sw_start(k, b):
        pltpu.async_copy(src_hbm.at[wid, k], SB[b], ES[b])
        pltpu.async_copy(w_hbm.at[wid, k], WB[b], ES[b])

    def sw_wait(k, b):
        pltpu.make_async_copy(src_hbm.at[wid, k], SB[b], ES[b]).wait()
        pltpu.make_async_copy(w_hbm.at[wid, k], WB[b], ES[b]).wait()

    def d_start(k, b):
        pltpu.async_copy(dst_hbm.at[wid, k], DB[b], DS[b])

    def d_wait(k, b):
        pltpu.make_async_copy(dst_hbm.at[wid, k], DB[b], DS[b]).wait()

    def g_start(k, b):
        pltpu.async_copy(x_hbm.at[SB[b].at[0]], R[b], GS[b])

    def g_wait(k, b):
        pltpu.make_async_copy(x_hbm.at[SB[b].at[0]], R[b], GS[b]).wait()

    def s_start(k, b):
        pltpu.async_copy(R[b], acc_sh.at[DB[b].at[0]], SS[b], add=True)

    def s_wait(k, b):
        pltpu.make_async_copy(R[b], acc_sh.at[DB[b].at[0]], SS[b]).wait()

    def mul(k, b):
        def mul_group(g, c2):
            wv = WB[b][0, pl.ds(g * 16, 16)]
            for j in range(16):
                wb = _lane_broadcast(wv, j)
                e = g * 16 + j
                for d in range(8):
                    sl = pl.ds(d * 16, 16)
                    R[b][e, sl] = R[b][e, sl] * wb
            return c2
        lax.fori_loop(0, CHUNK // 16, mul_group, 0)

    zv = jnp.zeros((16,), jnp.float32)

    def zfill(z, c2):
        for d in range(8):
            r3[z, pl.ds(d * 16, 16)] = zv
        return c2
    lax.fori_loop(0, CHUNK, zfill, 0)
    rows = s * 624
    for q in range(6):
        pltpu.sync_copy(r3, acc_sh.at[pl.ds(rows + q * CHUNK, CHUNK)])

    @pl.when(s < 15)
    def _():
        pltpu.sync_copy(r3.at[pl.ds(0, 48)],
                        acc_sh.at[pl.ds(rows + 6 * CHUNK, 48)])

    @pl.when(s == 15)
    def _():
        pltpu.sync_copy(r3.at[pl.ds(0, 64)],
                        acc_sh.at[pl.ds(rows + 6 * CHUNK, 64)])

    plsc.subcore_barrier()

    sw_start(0, 0)
    sw_start(1, 1)
    sw_start(2, 2)
    sw_start(3, 3)
    d_start(0, 0)
    d_start(1, 1)
    sw_wait(0, 0)
    g_start(0, 0)
    sw_wait(1, 1)
    g_start(1, 1)

    def outer(i, carry):
        k0 = i * NB
        for b in range(NB):
            k = k0 + b
            b2 = (b + 2) % NB
            g_wait(k, b)

            @pl.when(k >= 2)
            def _(k=k, b2=b2):
                s_wait(k - 2, b2)

            @pl.when(k < NCHUNKS - 2)
            def _(k=k, b2=b2):
                d_start(k + 2, b2)
                sw_wait(k + 2, b2)
                g_start(k + 2, b2)

            mul(k, b)

            @pl.when(k < NCHUNKS - NB)
            def _(k=k, b=b):
                sw_start(k + NB, b)

            d_wait(k, b)
            s_start(k, b)
        return carry

    lax.fori_loop(0, NCHUNKS // NB, outer, 0)
    s_wait(NCHUNKS - 2, (NCHUNKS - 2) % NB)
    s_wait(NCHUNKS - 1, (NCHUNKS - 1) % NB)

    plsc.subcore_barrier()

    @pl.when(s < 15)
    def _():
        pltpu.sync_copy(acc_sh.at[pl.ds(s * 624, 624)],
                        out_hbm.at[c, pl.ds(s * 624, 624)])

    @pl.when(s == 15)
    def _():
        pltpu.sync_copy(acc_sh.at[pl.ds(15 * 624, 640)],
                        out_hbm.at[c, pl.ds(15 * 624, 640)])


_agg_call = pl.kernel(
    _agg_body,
    out_type=jax.ShapeDtypeStruct((NC, N, D), jnp.float32),
    mesh=plsc.VectorSubcoreMesh(core_axis_name="c", subcore_axis_name="s"),
    scratch_types=(
        [pltpu.VMEM((1, CHUNK), jnp.int32) for _ in range(NB)]
        + [pltpu.VMEM((1, CHUNK), jnp.float32) for _ in range(NB)]
        + [pltpu.VMEM((1, CHUNK), jnp.int32) for _ in range(NB)]
        + [pltpu.VMEM((CHUNK, D), jnp.float32) for _ in range(NB)]
        + [pltpu.VMEM_SHARED((N, D), jnp.float32)]
        + [pltpu.SemaphoreType.DMA for _ in range(4 * NB)]
    ),
)


def _dense(p, W, b, relu):
    def body(p_ref, w_ref, b_ref, o_ref):
        acc = p_ref[0] + p_ref[1]
        r = jnp.dot(acc, w_ref[...], preferred_element_type=jnp.float32,
                    precision=lax.Precision.HIGHEST) + b_ref[...]
        o_ref[...] = jnp.maximum(r, 0.0) if relu else r

    R = 1000
    return pl.pallas_call(
        body,
        grid=(N // R,),
        in_specs=[
            pl.BlockSpec((2, R, D), lambda i: (0, i, 0)),
            pl.BlockSpec((D, D), lambda i: (0, 0)),
            pl.BlockSpec((1, D), lambda i: (0, 0)),
        ],
        out_specs=pl.BlockSpec((R, D), lambda i: (i, 0)),
        out_shape=jax.ShapeDtypeStruct((N, D), jnp.float32),
    )(p, W, b.reshape(1, D))


def kernel(x, edge_index, edge_weight, W1, b1, W2, b2):
    src = edge_index[0].astype(jnp.int32)
    dst = edge_index[1].astype(jnp.int32)
    w = edge_weight.astype(jnp.float32)
    pad = EP - E
    spread = (jnp.arange(pad, dtype=jnp.int32) * 37) % N
    src_p = jnp.concatenate([src, spread]).reshape(NW, NCHUNKS, 1, CHUNK)
    dst_p = jnp.concatenate([dst, spread]).reshape(NW, NCHUNKS, 1, CHUNK)
    w_p = jnp.pad(w, (0, pad)).reshape(NW, NCHUNKS, 1, CHUNK)

    p1 = _agg_call(x, src_p, dst_p, w_p)
    h = _dense(p1, W1, b1, relu=True)
    p2 = _agg_call(h, src_p, dst_p, w_p)
    return _dense(p2, W2, b2, relu=False)


# speedup vs baseline: 1.1939x; 1.1105x over previous
import jax
import jax.numpy as jnp
from jax import lax
from jax.experimental import pallas as pl
from jax.experimental.pallas import tpu as pltpu
from jax.experimental.pallas import tpu_sc as plsc

N = 10000
D = 128
E = 320000

NC = 2
NS = 16
NW = NC * NS

CHUNK = 80
NCHUNKS = 125
EPW = CHUNK * NCHUNKS
NB = 4
LAST = NCHUNKS - 1

_LANE_DNUMS = lax.GatherDimensionNumbers(
    offset_dims=(), collapsed_slice_dims=(0,), start_index_map=(0,))


def _lane_broadcast(vec, j):
    idx = jnp.full((16, 1), j, dtype=jnp.int32)
    return lax.gather(vec, idx, _LANE_DNUMS, (1,),
                      mode=lax.GatherScatterMode.PROMISE_IN_BOUNDS)


def _agg_body(x_hbm, src_hbm, dst_hbm, w_hbm, out_hbm,
              sb0, sb1, sb2, sb3, wb0, wb1, wb2, wb3,
              db0, db1, db2, db3, r0, r1, r2, r3, acc_sh,
              gs0, gs1, gs2, gs3, ss0, ss1, ss2, ss3,
              es0, es1, es2, es3, ds0, ds1, ds2, ds3):
    c = lax.axis_index("c")
    s = lax.axis_index("s")
    wid = s * NC + c
    ebase = wid * EPW
    SB = (sb0, sb1, sb2, sb3)
    WB = (wb0, wb1, wb2, wb3)
    DB = (db0, db1, db2, db3)
    R = (r0, r1, r2, r3)
    GS = (gs0, gs1, gs2, gs3)
    SS = (ss0, ss1, ss2, ss3)
    ES = (es0, es1, es2, es3)
    DS = (ds0, ds1, ds2, ds3)

    def sw_start(k, b):
        pltpu.async_copy(src_hbm.at[pl.ds(ebase + k * CHUNK, CHUNK)],
                         SB[b], ES[b])
        pltpu.async_copy(w_hbm.at[pl.ds(ebase + k * CHUNK, CHUNK)],
                         WB[b], ES[b])

    def sw_wait(k, b):
        pltpu.make_async_copy(src_hbm.at[pl.ds(ebase, CHUNK)],
                              SB[b], ES[b]).wait()
        pltpu.make_async_copy(w_hbm.at[pl.ds(ebase, CHUNK)],
                              WB[b], ES[b]).wait()

    def d_start(k, b):
        pltpu.async_copy(dst_hbm.at[pl.ds(ebase + k * CHUNK, CHUNK)],
                         DB[b], DS[b])

    def d_wait(k, b):
        pltpu.make_async_copy(dst_hbm.at[pl.ds(ebase, CHUNK)],
                              DB[b], DS[b]).wait()

    def g_start(k, b):
        pltpu.async_copy(x_hbm.at[SB[b]], R[b], GS[b])

    def g_wait(k, b):
        pltpu.make_async_copy(x_hbm.at[SB[b]], R[b], GS[b]).wait()

    def s_start(k, b):
        pltpu.async_copy(R[b], acc_sh.at[DB[b]], SS[b], add=True)

    def s_wait(k, b):
        pltpu.make_async_copy(R[b], acc_sh.at[DB[b]], SS[b]).wait()

    def mul(k, b):
        def mul_group(g, c2):
            wv = WB[b][pl.ds(g * 16, 16)]
            for j in range(16):
                wb = _lane_broadcast(wv, j)
                e = g * 16 + j
                for d in range(8):
                    sl = pl.ds(d * 16, 16)
                    R[b][e, sl] = R[b][e, sl] * wb
            return c2
        lax.fori_loop(0, CHUNK // 16, mul_group, 0)

    zv = jnp.zeros((16,), jnp.float32)

    def zfill(z, c2):
        for d in range(8):
            r3[z, pl.ds(d * 16, 16)] = zv
        return c2
    lax.fori_loop(0, CHUNK, zfill, 0)
    rows = s * 624
    for q in range(7):
        pltpu.sync_copy(r3, acc_sh.at[pl.ds(rows + q * CHUNK, CHUNK)])

    @pl.when(s < 15)
    def _():
        pltpu.sync_copy(r3.at[pl.ds(0, 64)],
                        acc_sh.at[pl.ds(rows + 7 * CHUNK, 64)])

    @pl.when(s == 15)
    def _():
        pltpu.sync_copy(r3, acc_sh.at[pl.ds(rows + 7 * CHUNK, CHUNK)])

    plsc.subcore_barrier()

    sw_start(0, 0)
    sw_start(1, 1)
    sw_start(2, 2)
    sw_start(3, 3)
    d_start(0, 0)
    d_start(1, 1)
    sw_wait(0, 0)
    g_start(0, 0)
    sw_wait(1, 1)
    g_start(1, 1)

    def step(k, b):
        b2 = (b + 2) % NB
        g_wait(k, b)

        @pl.when(k >= 2)
        def _(k=k, b2=b2):
            s_wait(k - 2, b2)

        @pl.when(k < NCHUNKS - 2)
        def _(k=k, b2=b2):
            d_start(k + 2, b2)
            sw_wait(k + 2, b2)
            g_start(k + 2, b2)

        mul(k, b)

        @pl.when(k < NCHUNKS - NB)
        def _(k=k, b=b):
            sw_start(k + NB, b)

        d_wait(k, b)
        s_start(k, b)

    def outer(i, carry):
        k0 = i * NB
        for b in range(NB):
            step(k0 + b, b)
        return carry

    lax.fori_loop(0, LAST // NB, outer, 0)
    step(LAST, LAST % NB)
    s_wait(NCHUNKS - 2, (NCHUNKS - 2) % NB)
    s_wait(NCHUNKS - 1, (NCHUNKS - 1) % NB)

    plsc.subcore_barrier()

    @pl.when(s < 15)
    def _():
        pltpu.sync_copy(acc_sh.at[pl.ds(s * 624, 624)],
                        out_hbm.at[c, pl.ds(s * 624, 624)])

    @pl.when(s == 15)
    def _():
        pltpu.sync_copy(acc_sh.at[pl.ds(15 * 624, 640)],
                        out_hbm.at[c, pl.ds(15 * 624, 640)])


_agg_call = pl.kernel(
    _agg_body,
    out_type=jax.ShapeDtypeStruct((NC, N, D), jnp.float32),
    mesh=plsc.VectorSubcoreMesh(core_axis_name="c", subcore_axis_name="s"),
    scratch_types=(
        [pltpu.VMEM((CHUNK,), jnp.int32) for _ in range(NB)]
        + [pltpu.VMEM((CHUNK,), jnp.float32) for _ in range(NB)]
        + [pltpu.VMEM((CHUNK,), jnp.int32) for _ in range(NB)]
        + [pltpu.VMEM((CHUNK, D), jnp.float32) for _ in range(NB)]
        + [pltpu.VMEM_SHARED((N, D), jnp.float32)]
        + [pltpu.SemaphoreType.DMA for _ in range(4 * NB)]
    ),
)


def _dense(p, W, b, relu):
    def body(p_ref, w_ref, b_ref, o_ref):
        acc = p_ref[0] + p_ref[1]
        r = jnp.dot(acc, w_ref[...], preferred_element_type=jnp.float32,
                    precision=lax.Precision.HIGHEST) + b_ref[...]
        o_ref[...] = jnp.maximum(r, 0.0) if relu else r

    R = 2000
    return pl.pallas_call(
        body,
        grid=(N // R,),
        in_specs=[
            pl.BlockSpec((2, R, D), lambda i: (0, i, 0)),
            pl.BlockSpec((D, D), lambda i: (0, 0)),
            pl.BlockSpec((1, D), lambda i: (0, 0)),
        ],
        out_specs=pl.BlockSpec((R, D), lambda i: (i, 0)),
        out_shape=jax.ShapeDtypeStruct((N, D), jnp.float32),
    )(p, W, b.reshape(1, D))


def kernel(x, edge_index, edge_weight, W1, b1, W2, b2):
    src = edge_index[0].astype(jnp.int32)
    dst = edge_index[1].astype(jnp.int32)
    w = edge_weight.astype(jnp.float32)

    p1 = _agg_call(x, src, dst, w)
    h = _dense(p1, W1, b1, relu=True)
    p2 = _agg_call(h, src, dst, w)
    return _dense(p2, W2, b2, relu=False)
